# baseline (speedup 1.0000x reference)
import functools

import jax
import jax.numpy as jnp
from jax import lax
from jax.experimental import pallas as pl
from jax.experimental.pallas import tpu as pltpu
from jax.experimental.pallas import tpu_sc as plsc

_NUM_CLASSES = 5
_EPS = 1e-08
_LANES = 128
_SENTINEL = 1 << 14

_NP = 100352
_R = _NP // _LANES
_CHUNK = 3136
_CROWS, _CCOLS = 28, 112
_HROWS = _NP // _CCOLS
_NCHUNKS = 32
_NBINS = 2064
_NB16 = _NBINS // 16


def _roll1_flat(a, col_is_last):
    t = jnp.roll(a, -1, axis=1)
    u = jnp.roll(t, -1, axis=0)
    return jnp.where(col_is_last, u, t)


def _pack_kernel(n, x_ref, c_ref, y_ref, key_ref, s_ref):
    planes = [x_ref[i] for i in range(_NUM_CLASSES)]
    m = planes[0]
    for p in planes[1:]:
        m = jnp.maximum(m, p)
    es = [jnp.exp(p - m) for p in planes]
    z = es[0]
    for e in es[1:]:
        z = z + e
    num = jnp.zeros_like(z)
    for i, e in enumerate(es):
        if i:
            num = num + e * jnp.float32(i)
    s = num / z

    col = lax.broadcasted_iota(jnp.int32, (_R, _LANES), 1)
    row = lax.broadcasted_iota(jnp.int32, (_R, _LANES), 0)
    fi = row * _LANES + col
    key = (c_ref[:] << 3) | y_ref[:]
    key_ref[:] = jnp.where(fi < n, key, _SENTINEL)
    s_ref[:] = s


def _group_kernel(keyf, svalf, keyg, svalg,
                  kv, sv, slots, h0, h1, cur, all_h, shared_h, sem):
    cid = lax.axis_index("c")
    tid = lax.axis_index("s")
    m = tid * 2 + cid
    lanes = lax.iota(jnp.int32, 16)
    idx_prev = jnp.maximum(lanes - 1, 0)
    idx_next = jnp.minimum(lanes + 1, 15)

    def ranks(bins):
        srt, orig = plsc.sort_key_val(bins, lanes)
        prev = srt[idx_prev]
        is_start = (lanes == 0) | (prev != srt)
        run_start = plsc.cummax(jnp.where(is_start, lanes, 0))
        rank = lanes - run_start
        nxt = srt[idx_next]
        is_end = (lanes == 15) | (nxt != srt)
        return srt, orig, rank, is_end

    def zero_body(j, _):
        z = jnp.zeros((16,), jnp.int32)
        h0[pl.ds(j * 16, 16)] = z
        h1[pl.ds(j * 16, 16)] = z
        return 0

    lax.fori_loop(0, _NB16, zero_body, 0)

    for sel, h in ((0, h0), (1, h1)):
        pltpu.sync_copy(keyf.at[pl.ds((tid * 2 + sel) * _CHUNK, _CHUNK)], kv)

        def hist_body(j, _, h=h):
            for t in range(_CCOLS // 16):
                bins = kv[pl.ds(j * _CCOLS + t * 16, 16)] >> 3
                srt, _orig, rank, is_end = ranks(bins)
                plsc.addupdate_scatter(h, [srt], rank + 1, mask=is_end)
            return 0


    pltpu.sync_copy(h0, shared_h.at[tid * 2])
    pltpu.sync_copy(h1, shared_h.at[tid * 2 + 1])
    plsc.subcore_barrier()
    pltpu.sync_copy(shared_h, all_h)

    def scan_body(j, carry):
        js = j * 16
        colsum = jnp.zeros((16,), jnp.int32)
        pre = jnp.zeros((16,), jnp.int32)
        for t in range(_NCHUNKS):
            rowv = all_h[t, pl.ds(js, 16)]
            colsum = colsum + rowv
            pre = pre + jnp.where(jnp.int32(t) < m, rowv, 0)
        incl = plsc.cumsum(colsum)
        cur[pl.ds(js, 16)] = carry + (incl - colsum) + pre
        return carry + jnp.max(incl)

    lax.fori_loop(0, _NB16, scan_body, jnp.int32(0))

    pltpu.sync_copy(keyf.at[pl.ds(m * _CHUNK, _CHUNK)], kv)
    pltpu.sync_copy(svalf.at[pl.ds(m * _CHUNK, _CHUNK)], sv)

    def slot_body(j, _):
        for t in range(_CCOLS // 16):
            bins = kv[pl.ds(j * _CCOLS + t * 16, 16)] >> 3
            srt, orig, rank, is_end = ranks(bins)
            base = plsc.load_gather(cur, [srt])
            rowv = jnp.full((16,), j, jnp.int32)
            plsc.store_scatter(slots, [rowv, orig + t * 16], base + rank)
            plsc.addupdate_scatter(cur, [srt], rank + 1, mask=is_end)
        return 0

    lax.fori_loop(0, _CROWS, slot_body, 0)

    copies = []
    for j in range(_CROWS):
        copies.append(pltpu.async_copy(
            kv.at[pl.ds(j * _CCOLS, _CCOLS)], keyg.at[slots.at[j]], sem))
        copies.append(pltpu.async_copy(
            sv.at[pl.ds(j * _CCOLS, _CCOLS)], svalg.at[slots.at[j]], sem))
    for c in copies:
        c.wait()


_group = functools.partial(
    pl.kernel,
    out_type=(
        jax.ShapeDtypeStruct((_NP,), jnp.int32),
        jax.ShapeDtypeStruct((_NP,), jnp.float32),
    ),
    scratch_types=[
        pltpu.VMEM((_CHUNK,), jnp.int32),
        pltpu.VMEM((_CHUNK,), jnp.float32),
        pltpu.VMEM((_CROWS, _CCOLS), jnp.int32),
        pltpu.VMEM((_NBINS,), jnp.int32),
        pltpu.VMEM((_NBINS,), jnp.int32),
        pltpu.VMEM((_NBINS,), jnp.int32),
        pltpu.VMEM((_NCHUNKS, _NBINS), jnp.int32),
        pltpu.VMEM_SHARED((_NCHUNKS, _NBINS), jnp.int32),
        pltpu.SemaphoreType.DMA,
    ],
    mesh=plsc.VectorSubcoreMesh(core_axis_name="c", subcore_axis_name="s"),
    compiler_params=pltpu.CompilerParams(
        needs_layout_passes=False, use_tc_tiling_on_sc=False),
)(_group_kernel)


def _loss_kernel(n, k_ref, s_ref, out_ref, kk_ref, sk_ref, acc_ref, wacc_ref):
    col = lax.broadcasted_iota(jnp.int32, (_R, _LANES), 1)
    row = lax.broadcasted_iota(jnp.int32, (_R, _LANES), 0)
    fi = row * _LANES + col
    col_last = col == (_LANES - 1)

    kk_ref[:] = _roll1_flat(k_ref[:], col_last)
    sk_ref[:] = _roll1_flat(s_ref[:], col_last)
    acc_ref[:] = jnp.zeros((_R, _LANES), jnp.float32)
    wacc_ref[:] = jnp.zeros((_R, _LANES), jnp.int32)

    def body(carry):
        k, _ = carry
        ks = k_ref[:]
        kk = kk_ref[:]
        sk = sk_ref[:]
        in_range = fi < (n - k)
        same = ((ks >> 3) == (kk >> 3)) & in_range
        yd = (ks & 7) - (kk & 7)
        active = same & (yd != 0)
        d = (s_ref[:] - sk) * jnp.sign(yd).astype(jnp.float32)
        loss = jnp.log1p(jnp.exp(-d))
        contrib = jnp.where(active, loss * jnp.abs(yd).astype(jnp.float32), 0.0)
        acc_ref[:] = acc_ref[:] + contrib
        wacc_ref[:] = wacc_ref[:] + active.astype(jnp.int32)
        kk_ref[:] = _roll1_flat(kk, col_last)
        sk_ref[:] = _roll1_flat(sk, col_last)
        return k + 1, jnp.any(same)

    lax.while_loop(lambda c: c[1], body, (jnp.int32(1), jnp.bool_(True)))

    total = jnp.sum(acc_ref[:])
    w = jnp.sum(wacc_ref[:]).astype(jnp.float32)
    out_ref[0, 0] = jnp.where(jnp.abs(w) < _EPS, 0.0, total / (w + _EPS))


@jax.jit
def kernel(inputs, targets, cluster_ids):
    n = targets.shape[0]

    xs = jnp.zeros((_NP, _NUM_CLASSES), jnp.float32).at[:n].set(
        inputs.astype(jnp.float32))
    x_planes = xs.T.reshape(_NUM_CLASSES, _R, _LANES)
    cs = jnp.zeros((_NP,), jnp.int32).at[:n].set(
        cluster_ids.astype(jnp.int32)).reshape(_R, _LANES)
    ys = jnp.zeros((_NP,), jnp.int32).at[:n].set(
        targets.astype(jnp.int32)).reshape(_R, _LANES)

    key, sval = pl.pallas_call(
        functools.partial(_pack_kernel, n),
        out_shape=(
            jax.ShapeDtypeStruct((_R, _LANES), jnp.int32),
            jax.ShapeDtypeStruct((_R, _LANES), jnp.float32),
        ),
        in_specs=[
            pl.BlockSpec(memory_space=pltpu.VMEM),
            pl.BlockSpec(memory_space=pltpu.VMEM),
            pl.BlockSpec(memory_space=pltpu.VMEM),
        ],
        out_specs=(
            pl.BlockSpec(memory_space=pltpu.VMEM),
            pl.BlockSpec(memory_space=pltpu.VMEM),
        ),
    )(x_planes, cs, ys)

    keyg, svalg = _group(key.reshape(_NP), sval.reshape(_NP))
    return jnp.sum(keyg).astype(jnp.float32) + jnp.sum(svalg)
    ks = keyg.reshape(_R, _LANES)
    ss = svalg.reshape(_R, _LANES)

    out = pl.pallas_call(
        functools.partial(_loss_kernel, n),
        out_shape=jax.ShapeDtypeStruct((1, 1), jnp.float32),
        in_specs=[
            pl.BlockSpec(memory_space=pltpu.VMEM),
            pl.BlockSpec(memory_space=pltpu.VMEM),
        ],
        out_specs=pl.BlockSpec(memory_space=pltpu.SMEM),
        scratch_shapes=[
            pltpu.VMEM((_R, _LANES), jnp.int32),
            pltpu.VMEM((_R, _LANES), jnp.float32),
            pltpu.VMEM((_R, _LANES), jnp.float32),
            pltpu.VMEM((_R, _LANES), jnp.int32),
        ],
    )(ks, ss)
    return out[0, 0]


# speedup vs baseline: 44.6538x; 44.6538x over previous
import functools

import jax
import jax.numpy as jnp
from jax import lax
from jax.experimental import pallas as pl
from jax.experimental.pallas import tpu as pltpu
from jax.experimental.pallas import tpu_sc as plsc

_NUM_CLASSES = 5
_EPS = 1e-08
_LANES = 128
_SENTINEL = 1 << 14

_NP = 100352
_R = _NP // _LANES
_CHUNK = 3136
_CROWS, _CCOLS = 28, 112
_HROWS = _NP // _CCOLS
_NCHUNKS = 32
_NBINS = 2064
_NB16 = _NBINS // 16


def _roll1_flat(a, col_is_last):
    t = jnp.roll(a, -1, axis=1)
    u = jnp.roll(t, -1, axis=0)
    return jnp.where(col_is_last, u, t)


def _pack_kernel(n, x_ref, c_ref, y_ref, key_ref, s_ref):
    planes = [x_ref[i] for i in range(_NUM_CLASSES)]
    m = planes[0]
    for p in planes[1:]:
        m = jnp.maximum(m, p)
    es = [jnp.exp(p - m) for p in planes]
    z = es[0]
    for e in es[1:]:
        z = z + e
    num = jnp.zeros_like(z)
    for i, e in enumerate(es):
        if i:
            num = num + e * jnp.float32(i)
    s = num / z

    col = lax.broadcasted_iota(jnp.int32, (_R, _LANES), 1)
    row = lax.broadcasted_iota(jnp.int32, (_R, _LANES), 0)
    fi = row * _LANES + col
    key = (c_ref[:] << 3) | y_ref[:]
    key_ref[:] = jnp.where(fi < n, key, _SENTINEL)
    s_ref[:] = s


def _group_kernel(keyf, svalf, keyg, svalg,
                  kv, sv, slots, h0, h1, cur, all_h, shared_h, sem):
    cid = lax.axis_index("c")
    tid = lax.axis_index("s")
    m = tid * 2 + cid
    lanes = lax.iota(jnp.int32, 16)
    idx_prev = jnp.maximum(lanes - 1, 0)
    idx_next = jnp.minimum(lanes + 1, 15)

    def ranks(bins):
        srt, orig = plsc.sort_key_val(bins, lanes)
        prev = srt[idx_prev]
        is_start = (lanes == 0) | (prev != srt)
        run_start = plsc.cummax(jnp.where(is_start, lanes, 0))
        rank = lanes - run_start
        nxt = srt[idx_next]
        is_end = (lanes == 15) | (nxt != srt)
        return srt, orig, rank, is_end

    def zero_body(j, _):
        z = jnp.zeros((16,), jnp.int32)
        h0[pl.ds(j * 16, 16)] = z
        h1[pl.ds(j * 16, 16)] = z
        return 0

    with jax.named_scope("p1_zero"):
        lax.fori_loop(0, _NB16, zero_body, 0)

    for sel, h in ((0, h0), (1, h1)):
        pltpu.sync_copy(keyf.at[pl.ds((tid * 2 + sel) * _CHUNK, _CHUNK)], kv)

        def hist_body(j, _, h=h):
            for t in range(_CCOLS // 16):
                bins = kv[pl.ds(j * _CCOLS + t * 16, 16)] >> 3
                srt, _orig, rank, is_end = ranks(bins)
                plsc.addupdate_scatter(h, [srt], rank + 1, mask=is_end)
            return 0

        with jax.named_scope("p1_hist"):
            lax.fori_loop(0, _CROWS, hist_body, 0)

    pltpu.sync_copy(h0, shared_h.at[tid * 2])
    pltpu.sync_copy(h1, shared_h.at[tid * 2 + 1])
    plsc.subcore_barrier()
    with jax.named_scope("p2_copy"):
        pltpu.sync_copy(shared_h, all_h)

    def scan_body(j, carry):
        js = j * 16
        colsum = jnp.zeros((16,), jnp.int32)
        pre = jnp.zeros((16,), jnp.int32)
        for t in range(_NCHUNKS):
            rowv = all_h[t, pl.ds(js, 16)]
            colsum = colsum + rowv
            pre = pre + jnp.where(jnp.int32(t) < m, rowv, 0)
        incl = plsc.cumsum(colsum)
        cur[pl.ds(js, 16)] = carry + (incl - colsum) + pre
        return carry + jnp.max(incl)

    with jax.named_scope("p2_scan"):
        lax.fori_loop(0, _NB16, scan_body, jnp.int32(0))

    pltpu.sync_copy(keyf.at[pl.ds(m * _CHUNK, _CHUNK)], kv)
    pltpu.sync_copy(svalf.at[pl.ds(m * _CHUNK, _CHUNK)], sv)

    def slot_body(j, _):
        for t in range(_CCOLS // 16):
            bins = kv[pl.ds(j * _CCOLS + t * 16, 16)] >> 3
            srt, orig, rank, is_end = ranks(bins)
            base = plsc.load_gather(cur, [srt])
            rowv = jnp.full((16,), j, jnp.int32)
            plsc.store_scatter(slots, [rowv, orig + t * 16], base + rank)
            plsc.addupdate_scatter(cur, [srt], rank + 1, mask=is_end)
        return 0

    with jax.named_scope("p3_slots"):
        lax.fori_loop(0, _CROWS, slot_body, 0)

    copies = []
    for j in range(_CROWS):
        copies.append(pltpu.async_copy(
            kv.at[pl.ds(j * _CCOLS, _CCOLS)], keyg.at[slots.at[j]], sem))
        copies.append(pltpu.async_copy(
            sv.at[pl.ds(j * _CCOLS, _CCOLS)], svalg.at[slots.at[j]], sem))
    with jax.named_scope("p3_dma"):
        for c in copies:
            c.wait()


_group = functools.partial(
    pl.kernel,
    out_type=(
        jax.ShapeDtypeStruct((_NP,), jnp.int32),
        jax.ShapeDtypeStruct((_NP,), jnp.float32),
    ),
    scratch_types=[
        pltpu.VMEM((_CHUNK,), jnp.int32),
        pltpu.VMEM((_CHUNK,), jnp.float32),
        pltpu.VMEM((_CROWS, _CCOLS), jnp.int32),
        pltpu.VMEM((_NBINS,), jnp.int32),
        pltpu.VMEM((_NBINS,), jnp.int32),
        pltpu.VMEM((_NBINS,), jnp.int32),
        pltpu.VMEM((_NCHUNKS, _NBINS), jnp.int32),
        pltpu.VMEM_SHARED((_NCHUNKS, _NBINS), jnp.int32),
        pltpu.SemaphoreType.DMA,
    ],
    mesh=plsc.VectorSubcoreMesh(core_axis_name="c", subcore_axis_name="s"),
    compiler_params=pltpu.CompilerParams(
        needs_layout_passes=False, use_tc_tiling_on_sc=False),
)(_group_kernel)


def _loss_kernel(n, k_ref, s_ref, out_ref, kk_ref, sk_ref, acc_ref, wacc_ref):
    col = lax.broadcasted_iota(jnp.int32, (_R, _LANES), 1)
    row = lax.broadcasted_iota(jnp.int32, (_R, _LANES), 0)
    fi = row * _LANES + col
    col_last = col == (_LANES - 1)

    kk_ref[:] = _roll1_flat(k_ref[:], col_last)
    sk_ref[:] = _roll1_flat(s_ref[:], col_last)
    acc_ref[:] = jnp.zeros((_R, _LANES), jnp.float32)
    wacc_ref[:] = jnp.zeros((_R, _LANES), jnp.int32)

    def body(carry):
        k, _ = carry
        ks = k_ref[:]
        kk = kk_ref[:]
        sk = sk_ref[:]
        in_range = fi < (n - k)
        same = ((ks >> 3) == (kk >> 3)) & in_range
        yd = (ks & 7) - (kk & 7)
        active = same & (yd != 0)
        d = (s_ref[:] - sk) * jnp.sign(yd).astype(jnp.float32)
        loss = jnp.log1p(jnp.exp(-d))
        contrib = jnp.where(active, loss * jnp.abs(yd).astype(jnp.float32), 0.0)
        acc_ref[:] = acc_ref[:] + contrib
        wacc_ref[:] = wacc_ref[:] + active.astype(jnp.int32)
        kk_ref[:] = _roll1_flat(kk, col_last)
        sk_ref[:] = _roll1_flat(sk, col_last)
        return k + 1, jnp.any(same)

    lax.while_loop(lambda c: c[1], body, (jnp.int32(1), jnp.bool_(True)))

    total = jnp.sum(acc_ref[:])
    w = jnp.sum(wacc_ref[:]).astype(jnp.float32)
    out_ref[0, 0] = jnp.where(jnp.abs(w) < _EPS, 0.0, total / (w + _EPS))


@jax.jit
def kernel(inputs, targets, cluster_ids):
    n = targets.shape[0]

    xs = jnp.zeros((_NP, _NUM_CLASSES), jnp.float32).at[:n].set(
        inputs.astype(jnp.float32))
    x_planes = xs.T.reshape(_NUM_CLASSES, _R, _LANES)
    cs = jnp.zeros((_NP,), jnp.int32).at[:n].set(
        cluster_ids.astype(jnp.int32)).reshape(_R, _LANES)
    ys = jnp.zeros((_NP,), jnp.int32).at[:n].set(
        targets.astype(jnp.int32)).reshape(_R, _LANES)

    key, sval = pl.pallas_call(
        functools.partial(_pack_kernel, n),
        out_shape=(
            jax.ShapeDtypeStruct((_R, _LANES), jnp.int32),
            jax.ShapeDtypeStruct((_R, _LANES), jnp.float32),
        ),
        in_specs=[
            pl.BlockSpec(memory_space=pltpu.VMEM),
            pl.BlockSpec(memory_space=pltpu.VMEM),
            pl.BlockSpec(memory_space=pltpu.VMEM),
        ],
        out_specs=(
            pl.BlockSpec(memory_space=pltpu.VMEM),
            pl.BlockSpec(memory_space=pltpu.VMEM),
        ),
    )(x_planes, cs, ys)

    keyg, svalg = _group(key.reshape(_NP), sval.reshape(_NP))
    ks = keyg.reshape(_R, _LANES)
    ss = svalg.reshape(_R, _LANES)

    out = pl.pallas_call(
        functools.partial(_loss_kernel, n),
        out_shape=jax.ShapeDtypeStruct((1, 1), jnp.float32),
        in_specs=[
            pl.BlockSpec(memory_space=pltpu.VMEM),
            pl.BlockSpec(memory_space=pltpu.VMEM),
        ],
        out_specs=pl.BlockSpec(memory_space=pltpu.SMEM),
        scratch_shapes=[
            pltpu.VMEM((_R, _LANES), jnp.int32),
            pltpu.VMEM((_R, _LANES), jnp.float32),
            pltpu.VMEM((_R, _LANES), jnp.float32),
            pltpu.VMEM((_R, _LANES), jnp.int32),
        ],
    )(ks, ss)
    return out[0, 0]


# speedup vs baseline: 116.5115x; 2.6092x over previous
import functools

import jax
import jax.numpy as jnp
from jax import lax
from jax.experimental import pallas as pl
from jax.experimental.pallas import tpu as pltpu
from jax.experimental.pallas import tpu_sc as plsc

_NUM_CLASSES = 5
_EPS = 1e-08
_LANES = 128
_SENTINEL = 1 << 14

_NP = 100352
_R = _NP // _LANES
_CHUNK = 3136
_CROWS, _CCOLS = 28, 112
_HROWS = _NP // _CCOLS
_NCHUNKS = 32
_NBINS = 2064
_HALF = _NP // 2
_NB16 = _NBINS // 16


def _roll1_flat(a, col_is_last):
    t = jnp.roll(a, -1, axis=1)
    u = jnp.roll(t, -1, axis=0)
    return jnp.where(col_is_last, u, t)


def _pack_kernel(n, x_ref, c_ref, y_ref, key_ref, s_ref):
    planes = [x_ref[i] for i in range(_NUM_CLASSES)]
    m = planes[0]
    for p in planes[1:]:
        m = jnp.maximum(m, p)
    es = [jnp.exp(p - m) for p in planes]
    z = es[0]
    for e in es[1:]:
        z = z + e
    num = jnp.zeros_like(z)
    for i, e in enumerate(es):
        if i:
            num = num + e * jnp.float32(i)
    s = num / z

    col = lax.broadcasted_iota(jnp.int32, (_R, _LANES), 1)
    row = lax.broadcasted_iota(jnp.int32, (_R, _LANES), 0)
    fi = row * _LANES + col
    key = (c_ref[:] << 3) | y_ref[:]
    key_ref[:] = jnp.where(fi < n, key, _SENTINEL)
    s_ref[:] = s


def _group_kernel(keyf, svalf, keyg, svalg,
                  kv, sv, slots, h0, h1, cur, all_h, shared_h,
                  outk_sh, outs_sh, sem):
    cid = lax.axis_index("c")
    tid = lax.axis_index("s")
    first_chunk = tid * 2
    lanes = lax.iota(jnp.int32, 16)
    idx_prev = jnp.maximum(lanes - 1, 0)
    idx_next = jnp.minimum(lanes + 1, 15)

    def ranks(bins):
        srt, orig = plsc.sort_key_val(bins, lanes)
        prev = srt[idx_prev]
        is_start = (lanes == 0) | (prev != srt)
        run_start = plsc.cummax(jnp.where(is_start, lanes, 0))
        rank = lanes - run_start
        nxt = srt[idx_next]
        is_end = (lanes == 15) | (nxt != srt)
        return srt, orig, rank, is_end

    def zero_body(j, _):
        z = jnp.zeros((16,), jnp.int32)
        h0[pl.ds(j * 16, 16)] = z
        h1[pl.ds(j * 16, 16)] = z
        return 0

    with jax.named_scope("p1_zero"):
        lax.fori_loop(0, _NB16, zero_body, 0)

    for sel, h in ((0, h0), (1, h1)):
        pltpu.sync_copy(keyf.at[pl.ds((tid * 2 + sel) * _CHUNK, _CHUNK)], kv)

        def hist_body(j, _, h=h):
            for t in range(_CCOLS // 16):
                bins = kv[pl.ds(j * _CCOLS + t * 16, 16)] >> 3
                srt, _orig, rank, is_end = ranks(bins)
                plsc.addupdate_scatter(h, [srt], rank + 1, mask=is_end)
            return 0

        with jax.named_scope("p1_hist"):
            lax.fori_loop(0, _CROWS, hist_body, 0)

    pltpu.sync_copy(h0, shared_h.at[tid * 2])
    pltpu.sync_copy(h1, shared_h.at[tid * 2 + 1])
    plsc.subcore_barrier()
    with jax.named_scope("p2_copy"):
        pltpu.sync_copy(shared_h, all_h)

    def scan_body(j, carry):
        js = j * 16
        colsum = jnp.zeros((16,), jnp.int32)
        pre = jnp.zeros((16,), jnp.int32)
        for t in range(_NCHUNKS):
            rowv = all_h[t, pl.ds(js, 16)]
            colsum = colsum + rowv
            pre = pre + jnp.where(jnp.int32(t) < first_chunk, rowv, 0)
        incl = plsc.cumsum(colsum)
        cur[pl.ds(js, 16)] = carry + (incl - colsum) + pre
        return carry + jnp.max(incl)

    with jax.named_scope("p2_scan"):
        lax.fori_loop(0, _NB16, scan_body, jnp.int32(0))

    for sel in (0, 1):
        mm = first_chunk + sel
        pltpu.sync_copy(keyf.at[pl.ds(mm * _CHUNK, _CHUNK)], kv)
        pltpu.sync_copy(svalf.at[pl.ds(mm * _CHUNK, _CHUNK)], sv)

        def slot_body(j, _):
            for t in range(_CCOLS // 16):
                bins = kv[pl.ds(j * _CCOLS + t * 16, 16)] >> 3
                srt, orig, rank, is_end = ranks(bins)
                base = plsc.load_gather(cur, [srt])
                slot = base + rank
                keep = jnp.where(cid == 0, slot < _HALF, slot >= _HALF)
                slotm = jnp.where(keep, slot, _NP + lanes)
                rowv = jnp.full((16,), j, jnp.int32)
                plsc.store_scatter(slots, [rowv, orig + t * 16], slotm)
                plsc.addupdate_scatter(cur, [srt], rank + 1, mask=is_end)
            return 0

        with jax.named_scope("p3_slots"):
            lax.fori_loop(0, _CROWS, slot_body, 0)

        copies = []
        for j in range(_CROWS):
            copies.append(pltpu.async_copy(
                kv.at[pl.ds(j * _CCOLS, _CCOLS)], outk_sh.at[slots.at[j]], sem))
            copies.append(pltpu.async_copy(
                sv.at[pl.ds(j * _CCOLS, _CCOLS)], outs_sh.at[slots.at[j]], sem))
        with jax.named_scope("p3_dma"):
            for c in copies:
                c.wait()

    plsc.subcore_barrier()
    off = cid * _HALF + tid * (_HALF // 16)
    with jax.named_scope("p3_out"):
        pltpu.sync_copy(outk_sh.at[pl.ds(off, _HALF // 16)],
                        keyg.at[pl.ds(off, _HALF // 16)])
        pltpu.sync_copy(outs_sh.at[pl.ds(off, _HALF // 16)],
                        svalg.at[pl.ds(off, _HALF // 16)])


_group = functools.partial(
    pl.kernel,
    out_type=(
        jax.ShapeDtypeStruct((_NP,), jnp.int32),
        jax.ShapeDtypeStruct((_NP,), jnp.float32),
    ),
    scratch_types=[
        pltpu.VMEM((_CHUNK,), jnp.int32),
        pltpu.VMEM((_CHUNK,), jnp.float32),
        pltpu.VMEM((_CROWS, _CCOLS), jnp.int32),
        pltpu.VMEM((_NBINS,), jnp.int32),
        pltpu.VMEM((_NBINS,), jnp.int32),
        pltpu.VMEM((_NBINS,), jnp.int32),
        pltpu.VMEM((_NCHUNKS, _NBINS), jnp.int32),
        pltpu.VMEM_SHARED((_NCHUNKS, _NBINS), jnp.int32),
        pltpu.VMEM_SHARED((_NP + 16,), jnp.int32),
        pltpu.VMEM_SHARED((_NP + 16,), jnp.float32),
        pltpu.SemaphoreType.DMA,
    ],
    mesh=plsc.VectorSubcoreMesh(core_axis_name="c", subcore_axis_name="s"),
    compiler_params=pltpu.CompilerParams(
        needs_layout_passes=False, use_tc_tiling_on_sc=False),
)(_group_kernel)


def _loss_kernel(n, k_ref, s_ref, out_ref, kk_ref, sk_ref, acc_ref, wacc_ref):
    col = lax.broadcasted_iota(jnp.int32, (_R, _LANES), 1)
    row = lax.broadcasted_iota(jnp.int32, (_R, _LANES), 0)
    fi = row * _LANES + col
    col_last = col == (_LANES - 1)

    kk_ref[:] = _roll1_flat(k_ref[:], col_last)
    sk_ref[:] = _roll1_flat(s_ref[:], col_last)
    acc_ref[:] = jnp.zeros((_R, _LANES), jnp.float32)
    wacc_ref[:] = jnp.zeros((_R, _LANES), jnp.int32)

    def body(carry):
        k, _ = carry
        ks = k_ref[:]
        kk = kk_ref[:]
        sk = sk_ref[:]
        in_range = fi < (n - k)
        same = ((ks >> 3) == (kk >> 3)) & in_range
        yd = (ks & 7) - (kk & 7)
        active = same & (yd != 0)
        d = (s_ref[:] - sk) * jnp.sign(yd).astype(jnp.float32)
        loss = jnp.log1p(jnp.exp(-d))
        contrib = jnp.where(active, loss * jnp.abs(yd).astype(jnp.float32), 0.0)
        acc_ref[:] = acc_ref[:] + contrib
        wacc_ref[:] = wacc_ref[:] + active.astype(jnp.int32)
        kk_ref[:] = _roll1_flat(kk, col_last)
        sk_ref[:] = _roll1_flat(sk, col_last)
        return k + 1, jnp.any(same)

    lax.while_loop(lambda c: c[1], body, (jnp.int32(1), jnp.bool_(True)))

    total = jnp.sum(acc_ref[:])
    w = jnp.sum(wacc_ref[:]).astype(jnp.float32)
    out_ref[0, 0] = jnp.where(jnp.abs(w) < _EPS, 0.0, total / (w + _EPS))


@jax.jit
def kernel(inputs, targets, cluster_ids):
    n = targets.shape[0]

    xs = jnp.zeros((_NP, _NUM_CLASSES), jnp.float32).at[:n].set(
        inputs.astype(jnp.float32))
    x_planes = xs.T.reshape(_NUM_CLASSES, _R, _LANES)
    cs = jnp.zeros((_NP,), jnp.int32).at[:n].set(
        cluster_ids.astype(jnp.int32)).reshape(_R, _LANES)
    ys = jnp.zeros((_NP,), jnp.int32).at[:n].set(
        targets.astype(jnp.int32)).reshape(_R, _LANES)

    key, sval = pl.pallas_call(
        functools.partial(_pack_kernel, n),
        out_shape=(
            jax.ShapeDtypeStruct((_R, _LANES), jnp.int32),
            jax.ShapeDtypeStruct((_R, _LANES), jnp.float32),
        ),
        in_specs=[
            pl.BlockSpec(memory_space=pltpu.VMEM),
            pl.BlockSpec(memory_space=pltpu.VMEM),
            pl.BlockSpec(memory_space=pltpu.VMEM),
        ],
        out_specs=(
            pl.BlockSpec(memory_space=pltpu.VMEM),
            pl.BlockSpec(memory_space=pltpu.VMEM),
        ),
    )(x_planes, cs, ys)

    keyg, svalg = _group(key.reshape(_NP), sval.reshape(_NP))
    ks = keyg.reshape(_R, _LANES)
    ss = svalg.reshape(_R, _LANES)

    out = pl.pallas_call(
        functools.partial(_loss_kernel, n),
        out_shape=jax.ShapeDtypeStruct((1, 1), jnp.float32),
        in_specs=[
            pl.BlockSpec(memory_space=pltpu.VMEM),
            pl.BlockSpec(memory_space=pltpu.VMEM),
        ],
        out_specs=pl.BlockSpec(memory_space=pltpu.SMEM),
        scratch_shapes=[
            pltpu.VMEM((_R, _LANES), jnp.int32),
            pltpu.VMEM((_R, _LANES), jnp.float32),
            pltpu.VMEM((_R, _LANES), jnp.float32),
            pltpu.VMEM((_R, _LANES), jnp.int32),
        ],
    )(ks, ss)
    return out[0, 0]
